# baseline (speedup 1.0000x reference)
import functools

import jax
import jax.numpy as jnp
from jax.experimental import pallas as pl
from jax.experimental.pallas import tpu as pltpu

_TT = 256


def _body(K, idx_ref, w_ref, xh0_ref, xh1_ref, *refs):
    del idx_ref
    o_ref = refs[-1]
    W_refs = refs[: 2 * K]
    b_refs = refs[2 * K : 3 * K]
    b = pl.program_id(0)
    Dh = W_refs[0].shape[2]
    for h, xh_ref in enumerate((xh0_ref, xh1_ref)):
        xh = xh_ref[0, 0].astype(jnp.bfloat16)
        acc = None
        for k in range(K):
            y = jnp.dot(
                xh[:, :Dh],
                W_refs[2 * k][0, 0].astype(jnp.bfloat16),
                preferred_element_type=jnp.float32,
            )
            y += jnp.dot(
                xh[:, Dh:],
                W_refs[2 * k + 1][0, 0].astype(jnp.bfloat16),
                preferred_element_type=jnp.float32,
            )
            y = jnp.maximum(y + b_refs[k][0], 0.0) * w_ref[b, k]
            acc = y if acc is None else acc + y
        o_ref[0, h] = acc


def kernel(x, w, idx, W_bank, b_bank):
    B, T, D = x.shape
    K = idx.shape[1]
    E = W_bank.shape[0]
    idx = idx.astype(jnp.int32)

    Th = T // 2
    Dh = D // 2
    x2 = x.reshape(B, 2, Th, D)
    W2 = W_bank.reshape(E, 2, Dh, D)

    def xh_map(h, b, t, idx_ref, w_ref):
        return (b, h, t, 0)

    def W_map(k, h, b, t, idx_ref, w_ref):
        return (idx_ref[b, k], h, 0, 0)

    def b_map(k, b, t, idx_ref, w_ref):
        return (idx_ref[b, k], 0, 0)

    def o_map(b, t, idx_ref, w_ref):
        return (b, 0, t, 0)

    in_specs = [
        pl.BlockSpec((1, 1, _TT, D), functools.partial(xh_map, h))
        for h in range(2)
    ]
    in_specs += [
        pl.BlockSpec((1, 1, Dh, D), functools.partial(W_map, k, h))
        for k in range(K)
        for h in range(2)
    ]
    in_specs += [
        pl.BlockSpec((1, 1, D), functools.partial(b_map, k)) for k in range(K)
    ]

    grid_spec = pltpu.PrefetchScalarGridSpec(
        num_scalar_prefetch=2,
        grid=(B, Th // _TT),
        in_specs=in_specs,
        out_specs=pl.BlockSpec((1, 2, _TT, D), o_map),
    )
    out = pl.pallas_call(
        functools.partial(_body, K),
        grid_spec=grid_spec,
        out_shape=jax.ShapeDtypeStruct((B, 2, Th, D), jnp.float32),
        compiler_params=pltpu.CompilerParams(
            dimension_semantics=("parallel", "arbitrary"),
        ),
    )(
        idx,
        w,
        x2,
        x2,
        *[W2] * (2 * K),
        *[b_bank.reshape(-1, 1, D)] * K,
    )
    return out.reshape(B, T, D)


# speedup vs baseline: 1.0799x; 1.0799x over previous
import functools

import jax
import jax.numpy as jnp
from jax.experimental import pallas as pl
from jax.experimental.pallas import tpu as pltpu

_TT = 512


def _body(K, idx_ref, w_ref, xh0_ref, xh1_ref, *refs):
    del idx_ref
    o_ref = refs[-1]
    W_refs = refs[:K]
    b_refs = refs[K : 2 * K]
    b = pl.program_id(0)
    for h, xh_ref in enumerate((xh0_ref, xh1_ref)):
        xh = xh_ref[0, 0].astype(jnp.bfloat16)
        acc = None
        for k in range(K):
            y = jnp.dot(
                xh,
                W_refs[k][0].astype(jnp.bfloat16),
                preferred_element_type=jnp.float32,
            )
            y = jnp.maximum(y + b_refs[k][0], 0.0) * w_ref[b, k]
            acc = y if acc is None else acc + y
        o_ref[0, h] = acc


def kernel(x, w, idx, W_bank, b_bank):
    B, T, D = x.shape
    K = idx.shape[1]
    E = W_bank.shape[0]
    idx = idx.astype(jnp.int32)

    Th = T // 2
    x2 = x.reshape(B, 2, Th, D)

    def xh_map(h, b, t, idx_ref, w_ref):
        return (b, h, t, 0)

    def W_map(k, b, t, idx_ref, w_ref):
        return (idx_ref[b, k], 0, 0)

    def b_map(k, b, t, idx_ref, w_ref):
        return (idx_ref[b, k], 0, 0)

    def o_map(b, t, idx_ref, w_ref):
        return (b, 0, t, 0)

    in_specs = [
        pl.BlockSpec((1, 1, _TT, D), functools.partial(xh_map, h))
        for h in range(2)
    ]
    in_specs += [
        pl.BlockSpec((1, D, D), functools.partial(W_map, k)) for k in range(K)
    ]
    in_specs += [
        pl.BlockSpec((1, 1, D), functools.partial(b_map, k)) for k in range(K)
    ]

    grid_spec = pltpu.PrefetchScalarGridSpec(
        num_scalar_prefetch=2,
        grid=(B, Th // _TT),
        in_specs=in_specs,
        out_specs=pl.BlockSpec((1, 2, _TT, D), o_map),
    )
    out = pl.pallas_call(
        functools.partial(_body, K),
        grid_spec=grid_spec,
        out_shape=jax.ShapeDtypeStruct((B, 2, Th, D), jnp.float32),
        compiler_params=pltpu.CompilerParams(
            dimension_semantics=("parallel", "arbitrary"),
        ),
    )(
        idx,
        w,
        x2,
        x2,
        *[W_bank] * K,
        *[b_bank.reshape(-1, 1, D)] * K,
    )
    return out.reshape(B, T, D)


# speedup vs baseline: 1.0851x; 1.0048x over previous
import functools

import jax
import jax.numpy as jnp
from jax.experimental import pallas as pl
from jax.experimental.pallas import tpu as pltpu

_TT = 512


def _body(K, idx_ref, w_ref, xh0_ref, xh1_ref, *refs):
    del idx_ref
    o_ref = refs[-1]
    W_refs = refs[:K]
    b_refs = refs[K : 2 * K]
    b = pl.program_id(0)
    for h, xh_ref in enumerate((xh0_ref, xh1_ref)):
        xh = xh_ref[0, 0].astype(jnp.bfloat16)
        acc = None
        for k in range(K):
            y = jnp.dot(
                xh,
                W_refs[k][0].astype(jnp.bfloat16),
                preferred_element_type=jnp.float32,
            )
            y = jnp.maximum(y + b_refs[k][0], 0.0) * w_ref[b, k]
            acc = y if acc is None else acc + y
        o_ref[0, h] = acc


def kernel(x, w, idx, W_bank, b_bank):
    B, T, D = x.shape
    K = idx.shape[1]
    E = W_bank.shape[0]
    idx = idx.astype(jnp.int32)

    Th = T // 2
    x2 = x.reshape(B, 2, Th, D)

    def xh_map(h, b, t, idx_ref, w_ref):
        return (b, h, t, 0)

    def W_map(k, b, t, idx_ref, w_ref):
        return (idx_ref[b, k], 0, 0)

    def b_map(k, b, t, idx_ref, w_ref):
        return (idx_ref[b, k], 0, 0)

    def o_map(b, t, idx_ref, w_ref):
        return (b, 0, t, 0)

    in_specs = [
        pl.BlockSpec((1, 1, _TT, D), functools.partial(xh_map, h))
        for h in range(2)
    ]
    in_specs += [
        pl.BlockSpec((1, D, D), functools.partial(W_map, k)) for k in range(K)
    ]
    in_specs += [
        pl.BlockSpec((1, 1, D), functools.partial(b_map, k)) for k in range(K)
    ]

    grid_spec = pltpu.PrefetchScalarGridSpec(
        num_scalar_prefetch=2,
        grid=(B, Th // _TT),
        in_specs=in_specs,
        out_specs=pl.BlockSpec((1, 2, _TT, D), o_map),
    )
    out = pl.pallas_call(
        functools.partial(_body, K),
        grid_spec=grid_spec,
        out_shape=jax.ShapeDtypeStruct((B, 2, Th, D), jnp.float32),
        compiler_params=pltpu.CompilerParams(
            dimension_semantics=("arbitrary", "arbitrary"),
        ),
    )(
        idx,
        w,
        x2,
        x2,
        *[W_bank] * K,
        *[b_bank.reshape(-1, 1, D)] * K,
    )
    return out.reshape(B, T, D)
